# baseline (speedup 1.0000x reference)
import jax
import jax.numpy as jnp
from jax import lax
from jax.experimental import pallas as pl

EMB = 256
MAXN = 2048
BSIZE = 8
LENGTHS = (512, 768, 1024, 1024, 1024, 1280, 1280, 1280)
TILE = 256
NBLK = tuple(l // TILE for l in LENGTHS)
CUMBLK = (0, 2, 5, 9, 13, 17, 22, 27)
BLK_PER_SEQ = MAXN // TILE
FEATD = 2 * EMB + 2


def _lookup(table, s):
    v = jnp.int32(table[-1])
    for i in range(len(table) - 2, -1, -1):
        v = jnp.where(s == i, jnp.int32(table[i]), v)
    return v


def _feat_index(s, j):
    return (_lookup(CUMBLK, s) + jnp.minimum(j, _lookup(NBLK, s) - 1), 0)


def _body(feat_ref, w6_ref, w7_ref, b6_ref, b7_ref, w5_ref, b5_ref,
          actor_ref, critic_ref):
    s = pl.program_id(0)
    j = pl.program_id(1)
    valid = j < _lookup(NBLK, s)

    @pl.when(valid)
    def _compute():
        x = feat_ref[...]
        dn = (((1,), (1,)), ((), ()))
        g = jnp.maximum(
            lax.dot_general(x[:, EMB:2 * EMB], w6_ref[...], dn,
                            preferred_element_type=jnp.float32) + b6_ref[...],
            0.0)
        l = jnp.maximum(
            lax.dot_general(x[:, :EMB], w7_ref[...], dn,
                            preferred_element_type=jnp.float32) + b7_ref[...],
            0.0)
        p = (lax.dot_general(g, w5_ref[:, :EMB], dn,
                             preferred_element_type=jnp.float32)
             + lax.dot_general(l, w5_ref[:, EMB:], dn,
                               preferred_element_type=jnp.float32)
             + b5_ref[...])
        actor_ref[...] = p[:, 0:1]
        q = jnp.where(x[:, 2 * EMB + 1] != 0.0, p[:, 1], -1e20)
        m = jnp.max(q)
        prev = jnp.where(j == 0, -jnp.inf, critic_ref[...])
        critic_ref[...] = jnp.maximum(prev, m)

    @pl.when(jnp.logical_not(valid))
    def _pad():
        actor_ref[...] = jnp.full((TILE, 1), -1e20, jnp.float32)


def kernel(features, W5pi, b5pi, W6pi, b6pi, W7pi, b7pi, W5v, b5v):
    b6r = b6pi.reshape(1, EMB)
    b7r = b7pi.reshape(1, EMB)
    w5 = jnp.concatenate([W5pi, W5v], axis=0)
    b5 = jnp.stack([b5pi, b5v], axis=1)

    actor_flat, crit3 = pl.pallas_call(
        _body,
        grid=(BSIZE, BLK_PER_SEQ),
        in_specs=[
            pl.BlockSpec((TILE, FEATD), _feat_index),
            pl.BlockSpec((EMB, EMB), lambda s, j: (0, 0)),
            pl.BlockSpec((EMB, EMB), lambda s, j: (0, 0)),
            pl.BlockSpec((1, EMB), lambda s, j: (0, 0)),
            pl.BlockSpec((1, EMB), lambda s, j: (0, 0)),
            pl.BlockSpec((2, 2 * EMB), lambda s, j: (0, 0)),
            pl.BlockSpec((1, 2), lambda s, j: (0, 0)),
        ],
        out_specs=[
            pl.BlockSpec((TILE, 1), lambda s, j: (s * BLK_PER_SEQ + j, 0)),
            pl.BlockSpec((1, 1, 1), lambda s, j: (s, 0, 0)),
        ],
        out_shape=[
            jax.ShapeDtypeStruct((BSIZE * MAXN, 1), jnp.float32),
            jax.ShapeDtypeStruct((BSIZE, 1, 1), jnp.float32),
        ],
    )(features, W6pi, W7pi, b6r, b7r, w5, b5)

    return actor_flat.reshape(BSIZE, MAXN, 1), crit3.reshape(BSIZE, 1)


# speedup vs baseline: 1.1692x; 1.1692x over previous
import jax
import jax.numpy as jnp
from jax.experimental import pallas as pl

EMB = 256
MAXN = 2048
BSIZE = 8
LENGTHS = (512, 768, 1024, 1024, 1024, 1280, 1280, 1280)
TILE = 256
NBLK = tuple(l // TILE for l in LENGTHS)
SEQ_OF_BLK = tuple(s for s in range(BSIZE) for _ in range(NBLK[s]))
JLOC_OF_BLK = tuple(j for s in range(BSIZE) for j in range(NBLK[s]))
NVALID = sum(NBLK)
FEATD = 2 * EMB + 2


def _lookup(table, i):
    v = jnp.int32(table[-1])
    for k in range(len(table) - 2, -1, -1):
        v = jnp.where(i == k, jnp.int32(table[k]), v)
    return v


def _body(feat_ref, w6t_ref, w7t_ref, b6_ref, b7_ref, w5a_ref, w5b_ref,
          b5_ref, actor_ref, critic_ref):
    i = pl.program_id(0)
    jloc = _lookup(JLOC_OF_BLK, i)
    first = jloc == 0

    @pl.when(first)
    def _fill():
        actor_ref[...] = jnp.full((MAXN, 1), -1e20, jnp.float32)

    x = feat_ref[...]
    g = jnp.maximum(
        jnp.dot(x[:, EMB:2 * EMB], w6t_ref[...],
                preferred_element_type=jnp.float32) + b6_ref[...], 0.0)
    l = jnp.maximum(
        jnp.dot(x[:, :EMB], w7t_ref[...],
                preferred_element_type=jnp.float32) + b7_ref[...], 0.0)
    p = (jnp.dot(g, w5a_ref[...], preferred_element_type=jnp.float32)
         + jnp.dot(l, w5b_ref[...], preferred_element_type=jnp.float32)
         + b5_ref[...])
    actor_ref[pl.ds(jloc * TILE, TILE), :] = p[:, 0:1]

    q = jnp.where(x[:, 2 * EMB + 1] != 0.0, p[:, 1], -1e20)
    m = jnp.max(q)
    prev = jnp.where(first, -jnp.inf, critic_ref[...])
    critic_ref[...] = jnp.maximum(prev, m)


def kernel(features, W5pi, b5pi, W6pi, b6pi, W7pi, b7pi, W5v, b5v):
    w6t = W6pi.T
    w7t = W7pi.T
    b6r = b6pi.reshape(1, EMB)
    b7r = b7pi.reshape(1, EMB)
    w5t = jnp.concatenate([W5pi, W5v], axis=0).T
    w5a = w5t[:EMB]
    w5b = w5t[EMB:]
    b5 = jnp.stack([b5pi, b5v], axis=1)

    actor_flat, crit3 = pl.pallas_call(
        _body,
        grid=(NVALID,),
        in_specs=[
            pl.BlockSpec((TILE, FEATD), lambda i: (i, 0)),
            pl.BlockSpec((EMB, EMB), lambda i: (0, 0)),
            pl.BlockSpec((EMB, EMB), lambda i: (0, 0)),
            pl.BlockSpec((1, EMB), lambda i: (0, 0)),
            pl.BlockSpec((1, EMB), lambda i: (0, 0)),
            pl.BlockSpec((EMB, 2), lambda i: (0, 0)),
            pl.BlockSpec((EMB, 2), lambda i: (0, 0)),
            pl.BlockSpec((1, 2), lambda i: (0, 0)),
        ],
        out_specs=[
            pl.BlockSpec((MAXN, 1), lambda i: (_lookup(SEQ_OF_BLK, i), 0)),
            pl.BlockSpec((1, 1, 1), lambda i: (_lookup(SEQ_OF_BLK, i), 0, 0)),
        ],
        out_shape=[
            jax.ShapeDtypeStruct((BSIZE * MAXN, 1), jnp.float32),
            jax.ShapeDtypeStruct((BSIZE, 1, 1), jnp.float32),
        ],
    )(features, w6t, w7t, b6r, b7r, w5a, w5b, b5)

    return actor_flat.reshape(BSIZE, MAXN, 1), crit3.reshape(BSIZE, 1)
